# Initial kernel scaffold; baseline (speedup 1.0000x reference)
import jax
import jax.numpy as jnp
from jax.experimental import pallas as pl


def kernel(input, table):
    raise NotImplementedError("write your pallas kernel here")


# speedup vs baseline: 3.2724x; 3.2724x over previous
import functools

import jax
import jax.numpy as jnp
from jax import lax
from jax.experimental import pallas as pl
from jax.experimental.pallas import tpu as pltpu, tpu_sc as plsc

NUM_ROWS = 100000
DIM = 128
TOTAL = 4096 * 50

_INFO = plsc.get_sparse_core_info()
_NC, _NS = _INFO.num_cores, _INFO.num_subcores
_NW = _NC * _NS
_B_PER_W = TOTAL // _NW
_CHUNK = 640
_NCHUNKS = _B_PER_W // _CHUNK


def _make_gather():
    mesh = plsc.VectorSubcoreMesh(core_axis_name="c", subcore_axis_name="s")

    @functools.partial(
        pl.kernel,
        out_type=jax.ShapeDtypeStruct((TOTAL, DIM), jnp.float32),
        mesh=mesh,
        scratch_types=[
            pltpu.VMEM((_CHUNK,), jnp.int32),
            pltpu.VMEM((_CHUNK, DIM), jnp.float32),
            pltpu.SemaphoreType.DMA,
        ],
    )
    def gather_kernel(idx_hbm, table_hbm, out_hbm, idx_v, rows_v, sem):
        wid = lax.axis_index("s") * _NC + lax.axis_index("c")
        base = wid * _B_PER_W
        for g in range(_NCHUNKS):
            off = base + g * _CHUNK
            pltpu.sync_copy(idx_hbm.at[pl.ds(off, _CHUNK)], idx_v)
            pltpu.async_copy(table_hbm.at[idx_v], rows_v, sem).wait()
            pltpu.sync_copy(rows_v, out_hbm.at[pl.ds(off, _CHUNK)])

    return gather_kernel


_gather = _make_gather()


@jax.jit
def kernel(input, table):
    flat_idx = input.reshape(-1).astype(jnp.int32)
    out = _gather(flat_idx, table)
    return out.reshape(input.shape + (DIM,))


# speedup vs baseline: 3.3203x; 1.0146x over previous
import functools

import jax
import jax.numpy as jnp
from jax import lax
from jax.experimental import pallas as pl
from jax.experimental.pallas import tpu as pltpu, tpu_sc as plsc

NUM_ROWS = 100000
DIM = 128
TOTAL = 4096 * 50

_INFO = plsc.get_sparse_core_info()
_NC, _NS = _INFO.num_cores, _INFO.num_subcores
_NW = _NC * _NS
_B_PER_W = TOTAL // _NW
_CHUNK = 400
_NCHUNKS = _B_PER_W // _CHUNK


def _make_gather():
    mesh = plsc.VectorSubcoreMesh(core_axis_name="c", subcore_axis_name="s")

    @functools.partial(
        pl.kernel,
        out_type=jax.ShapeDtypeStruct((TOTAL, DIM), jnp.float32),
        mesh=mesh,
        scratch_types=[
            pltpu.VMEM((_CHUNK,), jnp.int32),
            pltpu.VMEM((_CHUNK,), jnp.int32),
            pltpu.VMEM((_CHUNK, DIM), jnp.float32),
            pltpu.VMEM((_CHUNK, DIM), jnp.float32),
            pltpu.SemaphoreType.DMA,
            pltpu.SemaphoreType.DMA,
            pltpu.SemaphoreType.DMA,
            pltpu.SemaphoreType.DMA,
        ],
    )
    def gather_kernel(idx_hbm, table_hbm, out_hbm,
                      idx_v0, idx_v1, rows_v0, rows_v1,
                      gsem0, gsem1, wsem0, wsem1):
        wid = lax.axis_index("s") * _NC + lax.axis_index("c")
        base = wid * _B_PER_W
        idx_v = (idx_v0, idx_v1)
        rows_v = (rows_v0, rows_v1)
        gsem = (gsem0, gsem1)
        wsem = (wsem0, wsem1)

        def start_gather(g):
            b = g % 2
            off = base + g * _CHUNK
            pltpu.sync_copy(idx_hbm.at[pl.ds(off, _CHUNK)], idx_v[b])
            return pltpu.async_copy(table_hbm.at[idx_v[b]], rows_v[b], gsem[b])

        def start_write(g):
            b = g % 2
            off = base + g * _CHUNK
            return pltpu.async_copy(rows_v[b], out_hbm.at[pl.ds(off, _CHUNK)],
                                    wsem[b])

        gathers = [start_gather(0)]
        writes = [None] * _NCHUNKS
        for g in range(_NCHUNKS):
            if g + 1 < _NCHUNKS:
                if g >= 1:
                    writes[g - 1].wait()
                gathers.append(start_gather(g + 1))
            gathers[g].wait()
            writes[g] = start_write(g)
        writes[_NCHUNKS - 2].wait()
        writes[_NCHUNKS - 1].wait()

    return gather_kernel


_gather = _make_gather()


@jax.jit
def kernel(input, table):
    flat_idx = input.reshape(-1).astype(jnp.int32)
    out = _gather(flat_idx, table)
    return out.reshape(input.shape + (DIM,))


# speedup vs baseline: 5.8164x; 1.7518x over previous
import functools

import jax
import jax.numpy as jnp
from jax import lax
from jax.experimental import pallas as pl
from jax.experimental.pallas import tpu as pltpu, tpu_sc as plsc

NUM_ROWS = 100000
DIM = 128
TOTAL = 4096 * 50

_INFO = plsc.get_sparse_core_info()
_NC, _NS = _INFO.num_cores, _INFO.num_subcores
_NW = _NC * _NS
_B_PER_W = TOTAL // _NW
_CHUNK = 400
_NCHUNKS = _B_PER_W // _CHUNK


def _make_gather():
    mesh = plsc.VectorSubcoreMesh(core_axis_name="c", subcore_axis_name="s")

    @functools.partial(
        pl.kernel,
        out_type=jax.ShapeDtypeStruct((4096, 50, DIM), jnp.float32),
        mesh=mesh,
        scratch_types=[
            pltpu.VMEM((_CHUNK,), jnp.int32),
            pltpu.VMEM((_CHUNK,), jnp.int32),
            pltpu.VMEM((_CHUNK, DIM), jnp.float32),
            pltpu.VMEM((_CHUNK, DIM), jnp.float32),
            pltpu.SemaphoreType.DMA,
            pltpu.SemaphoreType.DMA,
            pltpu.SemaphoreType.DMA,
            pltpu.SemaphoreType.DMA,
        ],
    )
    def gather_kernel(idx_hbm, table_hbm, out_hbm,
                      idx_v0, idx_v1, rows_v0, rows_v1,
                      gsem0, gsem1, wsem0, wsem1):
        wid = lax.axis_index("s") * _NC + lax.axis_index("c")
        base = wid * _B_PER_W
        idx_v = (idx_v0, idx_v1)
        rows_v = (rows_v0, rows_v1)
        gsem = (gsem0, gsem1)
        wsem = (wsem0, wsem1)

        def start_gather(g):
            b = g % 2
            off = base + g * _CHUNK
            pltpu.sync_copy(idx_hbm.at[pl.ds(off, _CHUNK)], idx_v[b])
            return pltpu.async_copy(table_hbm.at[idx_v[b]], rows_v[b], gsem[b])

        def start_write(g):
            b = g % 2
            batch0 = (base + g * _CHUNK) // 50
            return [pltpu.async_copy(rows_v[b].at[pl.ds(j * 50, 50)],
                                     out_hbm.at[batch0 + j], wsem[b])
                    for j in range(_CHUNK // 50)]

        gathers = [start_gather(0)]
        writes = [None] * _NCHUNKS
        for g in range(_NCHUNKS):
            if g + 1 < _NCHUNKS:
                if g >= 1:
                    for d in writes[g - 1]:
                        d.wait()
                gathers.append(start_gather(g + 1))
            gathers[g].wait()
            writes[g] = start_write(g)
        for d in writes[_NCHUNKS - 2]:
            d.wait()
        for d in writes[_NCHUNKS - 1]:
            d.wait()

    return gather_kernel


_gather = _make_gather()


@jax.jit
def kernel(input, table):
    flat_idx = input.reshape(-1).astype(jnp.int32)
    return _gather(flat_idx, table)
